# Initial kernel scaffold; baseline (speedup 1.0000x reference)
import jax
import jax.numpy as jnp
from jax.experimental import pallas as pl


def kernel(inputs, weight):
    raise NotImplementedError("write your pallas kernel here")


# speedup vs baseline: 1.1233x; 1.1233x over previous
import functools

import jax
import jax.numpy as jnp
import numpy as np
from jax import lax
from jax.experimental import pallas as pl
from jax.experimental.pallas import tpu as pltpu
from jax.experimental.pallas import tpu_sc as plsc

NUM_EMBEDDINGS = 1000000
EMBEDDING_DIM = 32
_SCALE = float(np.sqrt(1.0 / NUM_EMBEDDINGS))

_NC = 2
_NS = 16
_NW = _NC * _NS
_LANES = 16
_SUB = 128


@functools.partial(jax.jit, static_argnums=(2, 3))
def _sc_embed(idx2d, weight, chunk, nchunks):
    n = idx2d.shape[0] * idx2d.shape[1]
    nsub = chunk // _SUB
    per_w = chunk * nchunks
    mesh = plsc.VectorSubcoreMesh(core_axis_name="c", subcore_axis_name="s")

    @functools.partial(
        pl.kernel,
        out_type=jax.ShapeDtypeStruct((n, EMBEDDING_DIM), jnp.float32),
        mesh=mesh,
        compiler_params=pltpu.CompilerParams(use_tc_tiling_on_sc=False),
        scratch_types=[
            pltpu.VMEM((nsub, _SUB), jnp.int32),
            pltpu.VMEM((chunk, EMBEDDING_DIM), jnp.float32),
            pltpu.SemaphoreType.DMA,
        ],
    )
    def body(idx_hbm, tab_hbm, out_hbm, idx_v, rows_v, sem):
        wid = lax.axis_index("s") * _NC + lax.axis_index("c")
        base = wid * per_w

        def chunk_body(ci, carry):
            off = pl.multiple_of(base + ci * chunk, chunk)
            irow = pl.multiple_of(off // _SUB, chunk // _SUB)
            pltpu.sync_copy(idx_hbm.at[pl.ds(irow, nsub)], idx_v)
            cps = [
                pltpu.async_copy(
                    tab_hbm.at[idx_v.at[j]],
                    rows_v.at[pl.ds(j * _SUB, _SUB)],
                    sem,
                )
                for j in range(nsub)
            ]
            for cp in cps:
                cp.wait()

            unroll = 8

            def mul_body(k, c2):
                r0 = k * unroll
                for u in range(unroll):
                    for h in range(EMBEDDING_DIM // _LANES):
                        sl = (r0 + u, pl.ds(h * _LANES, _LANES))
                        rows_v[sl] = rows_v[sl] * _SCALE
                return c2

            lax.fori_loop(0, chunk // unroll, mul_body, 0)
            pltpu.sync_copy(rows_v, out_hbm.at[pl.ds(off, chunk)])
            return carry

        lax.fori_loop(0, nchunks, chunk_body, 0)

    return body(idx2d, weight)


def kernel(inputs, weight):
    b, s = inputs.shape
    n = b * s
    idx = inputs.reshape(n // _SUB, _SUB).astype(jnp.int32)
    per_w = n // _NW
    chunk = 1024
    nchunks = per_w // chunk
    out = _sc_embed(idx, weight, chunk, nchunks)
    return out.reshape(b, s, EMBEDDING_DIM)


# speedup vs baseline: 1.1519x; 1.0254x over previous
import functools

import jax
import jax.numpy as jnp
import numpy as np
from jax import lax
from jax.experimental import pallas as pl
from jax.experimental.pallas import tpu as pltpu
from jax.experimental.pallas import tpu_sc as plsc

NUM_EMBEDDINGS = 1000000
EMBEDDING_DIM = 32
_SCALE = float(np.sqrt(1.0 / NUM_EMBEDDINGS))

_NC = 2
_NS = 16
_NW = _NC * _NS
_LANES = 16
_SUB = 128


@functools.partial(jax.jit, static_argnums=(2, 3))
def _sc_embed(idx2d, weight, chunk, nchunks):
    n = idx2d.shape[0] * idx2d.shape[1]
    nsub = chunk // _SUB
    per_w = chunk * nchunks
    mesh = plsc.VectorSubcoreMesh(core_axis_name="c", subcore_axis_name="s")

    @functools.partial(
        pl.kernel,
        out_type=jax.ShapeDtypeStruct((n, EMBEDDING_DIM), jnp.float32),
        mesh=mesh,
        compiler_params=pltpu.CompilerParams(use_tc_tiling_on_sc=False),
        scratch_types=[
            pltpu.VMEM((2, nsub, _SUB), jnp.int32),
            pltpu.VMEM((2, chunk, EMBEDDING_DIM), jnp.float32),
            pltpu.SemaphoreType.DMA,
            pltpu.SemaphoreType.DMA,
        ],
    )
    def body(idx_hbm, tab_hbm, out_hbm, idx_v, rows_v, sem0, sem1):
        sems = (sem0, sem1)
        wid = lax.axis_index("s") * _NC + lax.axis_index("c")
        base = wid * per_w

        def gathers(buf):
            return [
                pltpu.make_async_copy(
                    tab_hbm.at[idx_v.at[buf].at[j]],
                    rows_v.at[buf].at[pl.ds(j * _SUB, _SUB)],
                    sems[buf],
                )
                for j in range(nsub)
            ]

        def load_and_fire(ci, buf):
            off = pl.multiple_of(base + ci * chunk, chunk)
            irow = pl.multiple_of(off // _SUB, nsub)
            pltpu.sync_copy(idx_hbm.at[pl.ds(irow, nsub)], idx_v.at[buf])
            for cp in gathers(buf):
                cp.start()

        unroll = 8

        def scale(buf):
            def mul_body(k, c2):
                r0 = k * unroll
                for u in range(unroll):
                    for h in range(EMBEDDING_DIM // _LANES):
                        sl = (buf, r0 + u, pl.ds(h * _LANES, _LANES))
                        rows_v[sl] = rows_v[sl] * _SCALE
                return c2

            lax.fori_loop(0, chunk // unroll, mul_body, 0)

        load_and_fire(0, 0)
        for ci in range(nchunks):
            buf = ci % 2
            if ci + 1 < nchunks:
                load_and_fire(ci + 1, 1 - buf)
            for cp in gathers(buf):
                cp.wait()
            scale(buf)
            off = pl.multiple_of(base + ci * chunk, chunk)
            pltpu.sync_copy(rows_v.at[buf], out_hbm.at[pl.ds(off, chunk)])

    return body(idx2d, weight)


def kernel(inputs, weight):
    b, s = inputs.shape
    n = b * s
    idx = inputs.reshape(n // _SUB, _SUB).astype(jnp.int32)
    per_w = n // _NW
    chunk = 1024
    nchunks = per_w // chunk
    out = _sc_embed(idx, weight, chunk, nchunks)
    return out.reshape(b, s, EMBEDDING_DIM)


# speedup vs baseline: 1.1524x; 1.0004x over previous
import functools

import jax
import jax.numpy as jnp
import numpy as np
from jax import lax
from jax.experimental import pallas as pl
from jax.experimental.pallas import tpu as pltpu
from jax.experimental.pallas import tpu_sc as plsc

NUM_EMBEDDINGS = 1000000
EMBEDDING_DIM = 32
_SCALE = float(np.sqrt(1.0 / NUM_EMBEDDINGS))

_NC = 2
_NS = 16
_NW = _NC * _NS
_LANES = 16
_SUB = 128


@functools.partial(jax.jit, static_argnums=(2, 3))
def _sc_embed(idx2d, weight, chunk, nchunks):
    n = idx2d.shape[0] * idx2d.shape[1]
    nsub = chunk // _SUB
    per_w = chunk * nchunks
    mesh = plsc.VectorSubcoreMesh(core_axis_name="c", subcore_axis_name="s")

    @functools.partial(
        pl.kernel,
        out_type=jax.ShapeDtypeStruct((n, EMBEDDING_DIM), jnp.float32),
        mesh=mesh,
        compiler_params=pltpu.CompilerParams(use_tc_tiling_on_sc=False),
        scratch_types=[
            pltpu.VMEM((3, nsub, _SUB), jnp.int32),
            pltpu.VMEM((3, chunk, EMBEDDING_DIM), jnp.float32),
            pltpu.SemaphoreType.DMA,
            pltpu.SemaphoreType.DMA,
            pltpu.SemaphoreType.DMA,
        ],
    )
    def body(idx_hbm, tab_hbm, out_hbm, idx_v, rows_v, sem0, sem1, sem2):
        sems = (sem0, sem1, sem2)
        wid = lax.axis_index("s") * _NC + lax.axis_index("c")
        base = wid * per_w

        def gathers(buf):
            return [
                pltpu.make_async_copy(
                    tab_hbm.at[idx_v.at[buf].at[j]],
                    rows_v.at[buf].at[pl.ds(j * _SUB, _SUB)],
                    sems[buf],
                )
                for j in range(nsub)
            ]

        def load_and_fire(ci, buf):
            off = pl.multiple_of(base + ci * chunk, chunk)
            irow = pl.multiple_of(off // _SUB, nsub)
            pltpu.sync_copy(idx_hbm.at[pl.ds(irow, nsub)], idx_v.at[buf])
            for cp in gathers(buf):
                cp.start()

        unroll = 8

        def scale(buf):
            def mul_body(k, c2):
                r0 = k * unroll
                for u in range(unroll):
                    for h in range(EMBEDDING_DIM // _LANES):
                        sl = (buf, r0 + u, pl.ds(h * _LANES, _LANES))
                        rows_v[sl] = rows_v[sl] * _SCALE
                return c2

            lax.fori_loop(0, chunk // unroll, mul_body, 0)

        load_and_fire(0, 0)
        load_and_fire(1, 1)
        for ci in range(nchunks):
            buf = ci % 3
            if ci + 2 < nchunks:
                load_and_fire(ci + 2, (ci + 2) % 3)
            for cp in gathers(buf):
                cp.wait()
            scale(buf)
            off = pl.multiple_of(base + ci * chunk, chunk)
            pltpu.sync_copy(rows_v.at[buf], out_hbm.at[pl.ds(off, chunk)])

    return body(idx2d, weight)


def kernel(inputs, weight):
    b, s = inputs.shape
    n = b * s
    idx = inputs.reshape(n // _SUB, _SUB).astype(jnp.int32)
    per_w = n // _NW
    chunk = 1024
    nchunks = per_w // chunk
    out = _sc_embed(idx, weight, chunk, nchunks)
    return out.reshape(b, s, EMBEDDING_DIM)


# speedup vs baseline: 1.1535x; 1.0009x over previous
import functools

import jax
import jax.numpy as jnp
import numpy as np
from jax import lax
from jax.experimental import pallas as pl
from jax.experimental.pallas import tpu as pltpu
from jax.experimental.pallas import tpu_sc as plsc

NUM_EMBEDDINGS = 1000000
EMBEDDING_DIM = 32
_SCALE = float(np.sqrt(1.0 / NUM_EMBEDDINGS))

_NC = 2
_NS = 16
_NW = _NC * _NS
_LANES = 16
_SUB = 128


@functools.partial(jax.jit, static_argnums=(2, 3))
def _sc_embed(idx1d, weight, chunk, nchunks):
    n = idx1d.shape[0]
    per_w = chunk * nchunks
    mesh = plsc.VectorSubcoreMesh(core_axis_name="c", subcore_axis_name="s")

    @functools.partial(
        pl.kernel,
        out_type=jax.ShapeDtypeStruct((n, EMBEDDING_DIM), jnp.float32),
        mesh=mesh,
        compiler_params=pltpu.CompilerParams(use_tc_tiling_on_sc=False),
        scratch_types=[
            pltpu.VMEM((3, chunk), jnp.int32),
            pltpu.VMEM((3, chunk, EMBEDDING_DIM), jnp.float32),
            pltpu.SemaphoreType.DMA,
            pltpu.SemaphoreType.DMA,
            pltpu.SemaphoreType.DMA,
        ],
    )
    def body(idx_hbm, tab_hbm, out_hbm, idx_v, rows_v, sem0, sem1, sem2):
        sems = (sem0, sem1, sem2)
        wid = lax.axis_index("s") * _NC + lax.axis_index("c")
        base = wid * per_w

        def gathers(buf):
            return [
                pltpu.make_async_copy(
                    tab_hbm.at[idx_v.at[buf]],
                    rows_v.at[buf],
                    sems[buf],
                )
            ]

        def load_and_fire(ci, buf):
            off = pl.multiple_of(base + ci * chunk, chunk)
            pltpu.sync_copy(idx_hbm.at[pl.ds(off, chunk)], idx_v.at[buf])
            for cp in gathers(buf):
                cp.start()

        unroll = 8

        def scale(buf):
            def mul_body(k, c2):
                r0 = k * unroll
                for u in range(unroll):
                    for h in range(EMBEDDING_DIM // _LANES):
                        sl = (buf, r0 + u, pl.ds(h * _LANES, _LANES))
                        rows_v[sl] = rows_v[sl] * _SCALE
                return c2

            lax.fori_loop(0, chunk // unroll, mul_body, 0)

        load_and_fire(0, 0)
        load_and_fire(1, 1)
        for ci in range(nchunks):
            buf = ci % 3
            if ci + 2 < nchunks:
                load_and_fire(ci + 2, (ci + 2) % 3)
            for cp in gathers(buf):
                cp.wait()
            scale(buf)
            off = pl.multiple_of(base + ci * chunk, chunk)
            pltpu.sync_copy(rows_v.at[buf], out_hbm.at[pl.ds(off, chunk)])

    return body(idx1d, weight)


def kernel(inputs, weight):
    b, s = inputs.shape
    n = b * s
    idx = inputs.reshape(n).astype(jnp.int32)
    per_w = n // _NW
    chunk = 1024
    nchunks = per_w // chunk
    out = _sc_embed(idx, weight, chunk, nchunks)
    return out.reshape(b, s, EMBEDDING_DIM)


# speedup vs baseline: 1.8518x; 1.6054x over previous
import functools

import jax
import jax.numpy as jnp
import numpy as np
from jax import lax
from jax.experimental import pallas as pl
from jax.experimental.pallas import tpu as pltpu
from jax.experimental.pallas import tpu_sc as plsc

NUM_EMBEDDINGS = 1000000
EMBEDDING_DIM = 32
_SCALE = float(np.sqrt(1.0 / NUM_EMBEDDINGS))

_NC = 2
_NS = 16
_NW = _NC * _NS
_LANES = 16
_CROWS = 16


@functools.partial(jax.jit, static_argnums=(2,))
def _sc_embed(inputs, weight, nchunks):
    b, s = inputs.shape
    rows_per_w = b // _NW
    mesh = plsc.VectorSubcoreMesh(core_axis_name="c", subcore_axis_name="s")

    @functools.partial(
        pl.kernel,
        out_type=jax.ShapeDtypeStruct((b, s, EMBEDDING_DIM), jnp.float32),
        mesh=mesh,
        compiler_params=pltpu.CompilerParams(use_tc_tiling_on_sc=False),
        scratch_types=[
            pltpu.VMEM((_CROWS, s), jnp.int32),
            pltpu.VMEM((_CROWS, s), jnp.int32),
            pltpu.VMEM((_CROWS, s, EMBEDDING_DIM), jnp.float32),
            pltpu.VMEM((_CROWS, s, EMBEDDING_DIM), jnp.float32),
            pltpu.SemaphoreType.DMA,
            pltpu.SemaphoreType.DMA,
        ],
    )
    def body(in_hbm, tab_hbm, out_hbm, idx0, idx1, rows0, rows1, sem0, sem1):
        idxs = (idx0, idx1)
        rows = (rows0, rows1)
        sems = (sem0, sem1)
        wid = lax.axis_index("s") * _NC + lax.axis_index("c")
        base = wid * rows_per_w

        def gathers(buf):
            return [
                pltpu.make_async_copy(
                    tab_hbm.at[idxs[buf].at[r]],
                    rows[buf].at[r],
                    sems[buf],
                )
                for r in range(_CROWS)
            ]

        def load_and_fire(ci, buf):
            row0 = pl.multiple_of(base + ci * _CROWS, _CROWS)
            pltpu.sync_copy(in_hbm.at[pl.ds(row0, _CROWS)], idxs[buf])
            for cp in gathers(buf):
                cp.start()

        def scale(buf):
            def mul_body(r, c2):
                for q in range(s):
                    for h in range(EMBEDDING_DIM // _LANES):
                        sl = (r, q, pl.ds(h * _LANES, _LANES))
                        rows[buf][sl] = rows[buf][sl] * _SCALE
                return c2

            lax.fori_loop(0, _CROWS, mul_body, 0)

        def process(ci, buf):
            @pl.when(ci + 1 < nchunks)
            def _():
                load_and_fire(ci + 1, 1 - buf)

            for cp in gathers(buf):
                cp.wait()
            scale(buf)
            row0 = pl.multiple_of(base + ci * _CROWS, _CROWS)
            pltpu.sync_copy(rows[buf], out_hbm.at[pl.ds(row0, _CROWS)])

        load_and_fire(0, 0)

        def group(g, carry):
            process(g * 2, 0)
            process(g * 2 + 1, 1)
            return carry

        lax.fori_loop(0, nchunks // 2, group, 0)

    return body(inputs, weight)


def kernel(inputs, weight):
    b, s = inputs.shape
    nchunks = b // _NW // _CROWS
    return _sc_embed(inputs, weight, nchunks)
